# baseline (speedup 1.0000x reference)
import functools

import jax
import jax.numpy as jnp
from jax import lax
from jax.experimental import pallas as pl
from jax.experimental.pallas import tpu as pltpu
from jax.experimental.pallas import tpu_sc as plsc

NC = 2
NS = 16
NW = NC * NS
LANES = 16
EB = 112
ZC = 80


def _sc_spmm(src, edata, vdata, nb1, n_pad, d):
  nb0 = edata.shape[1]
  rows_per_tile = n_pad // NS
  assert rows_per_tile % ZC == 0
  mesh = plsc.VectorSubcoreMesh(core_axis_name="c", subcore_axis_name="s")

  @functools.partial(
      pl.kernel,
      out_type=jax.ShapeDtypeStruct((NC, n_pad, d), jnp.float32),
      mesh=mesh,
      scratch_types=[
          pltpu.VMEM((3, 2, EB), jnp.int32),
          pltpu.VMEM((3, 1, EB), jnp.float32),
          pltpu.VMEM((3, EB), jnp.int32),
          pltpu.VMEM((3, EB, d), jnp.float32),
          pltpu.VMEM_SHARED((n_pad, d), jnp.float32),
          pltpu.SemaphoreType.DMA,
          pltpu.SemaphoreType.DMA,
          pltpu.SemaphoreType.DMA,
          pltpu.SemaphoreType.DMA,
          pltpu.SemaphoreType.DMA,
          pltpu.SemaphoreType.DMA,
          pltpu.SemaphoreType.DMA,
      ],
  )
  def k(src_hbm, edata_hbm, vdata_hbm, out_hbm, ebuf, vbuf, ridx, rbuf,
        acc_sh, sem_i, sg0, sg1, sg2, ss0, ss1, ss2):
    sem_g = (sg0, sg1, sg2)
    sem_s = (ss0, ss1, ss2)
    cid = lax.axis_index("c")
    sid = lax.axis_index("s")
    wid = sid * NC + cid
    cnt = jnp.where(cid == 0, nb0, nb1)

    for b in range(2):
      pltpu.sync_copy(edata_hbm.at[wid, b], ebuf.at[b])
      pltpu.sync_copy(vdata_hbm.at[wid, b], vbuf.at[b])
      pltpu.async_copy(src_hbm.at[ebuf.at[b, 0]], rbuf.at[b], sem_g[b])

    def zfill(r, _):
      for cb in range(d // LANES):
        rbuf[2, r, pl.ds(cb * LANES, LANES)] = jnp.zeros((LANES,), jnp.float32)
      return 0
    lax.fori_loop(0, ZC, zfill, 0)

    def zcopy(i, _):
      pltpu.sync_copy(
          rbuf.at[2, pl.ds(0, ZC)],
          acc_sh.at[pl.ds(sid * rows_per_tile + i * ZC, ZC)])
      return 0
    lax.fori_loop(0, rows_per_tile // ZC, zcopy, 0)
    plsc.subcore_barrier()

    def tri(t3, _):
      for b in range(3):
        t = t3 * 3 + b
        p = b
        f = (b + 2) % 3
        has_next = t + 2 < cnt

        @pl.when(has_next)
        def _():
          pltpu.async_copy(edata_hbm.at[wid, t + 2], ebuf.at[f], sem_i)
          pltpu.async_copy(vdata_hbm.at[wid, t + 2], vbuf.at[f], sem_i)

        pltpu.make_async_copy(
            src_hbm.at[ebuf.at[p, 0]], rbuf.at[p], sem_g[p]).wait()

        def scale(g, _):
          sl16 = pl.ds(g * LANES, LANES)
          ridx[p, sl16] = ebuf[p, 1, sl16]
          vv = vbuf[p, 0, sl16]
          for i in range(LANES):
            e = g * LANES + i
            v = vv[i]
            for cb in range(d // LANES):
              sl = pl.ds(cb * LANES, LANES)
              rbuf[p, e, sl] = rbuf[p, e, sl] * v
          return 0
        lax.fori_loop(0, EB // LANES, scale, 0)

        pltpu.async_copy(rbuf.at[p], acc_sh.at[ridx.at[p]], sem_s[p],
                         add=True)

        @pl.when(t >= 1)
        def _():
          pltpu.make_async_copy(
              rbuf.at[f], acc_sh.at[ridx.at[f]], sem_s[f]).wait()

        @pl.when(has_next)
        def _():
          pltpu.make_async_copy(
              edata_hbm.at[wid, t + 2], ebuf.at[f], sem_i).wait()
          pltpu.make_async_copy(
              vdata_hbm.at[wid, t + 2], vbuf.at[f], sem_i).wait()
          pltpu.async_copy(src_hbm.at[ebuf.at[f, 0]], rbuf.at[f], sem_g[f])
      return 0
    lax.fori_loop(0, cnt // 3, tri, 0)

    pltpu.make_async_copy(
        rbuf.at[2], acc_sh.at[ridx.at[2]], sem_s[2]).wait()
    plsc.subcore_barrier()

    pltpu.sync_copy(
        acc_sh.at[pl.ds(sid * rows_per_tile, rows_per_tile)],
        out_hbm.at[cid, pl.ds(sid * rows_per_tile, rows_per_tile)])

  return k(src, edata, vdata)


def _tc_add(a, b, n):
  d = a.shape[1]
  blk = n // 5

  def body(a_ref, b_ref, o_ref):
    o_ref[...] = a_ref[...] + b_ref[...]

  return pl.pallas_call(
      body,
      grid=(5,),
      in_specs=[pl.BlockSpec((blk, d), lambda i: (i, 0)),
                pl.BlockSpec((blk, d), lambda i: (i, 0))],
      out_specs=pl.BlockSpec((blk, d), lambda i: (i, 0)),
      out_shape=jax.ShapeDtypeStruct((n, d), jnp.float32),
  )(a, b)


def _tc_addmm(a, b, w, n):
  d = a.shape[1]
  blk = n // 5

  def body(a_ref, b_ref, w_ref, o_ref):
    o_ref[...] = jnp.dot(a_ref[...] + b_ref[...], w_ref[...],
                         preferred_element_type=jnp.float32)

  return pl.pallas_call(
      body,
      grid=(5,),
      in_specs=[pl.BlockSpec((blk, d), lambda i: (i, 0)),
                pl.BlockSpec((blk, d), lambda i: (i, 0)),
                pl.BlockSpec((d, d), lambda i: (0, 0))],
      out_specs=pl.BlockSpec((blk, d), lambda i: (i, 0)),
      out_shape=jax.ShapeDtypeStruct((n, d), jnp.float32),
  )(a, b, w)


def _pack(flat, nb0, nb1):
  cap0 = NS * nb0 * EB
  part0 = flat[:cap0].reshape(NS, nb0, EB)
  part1 = flat[cap0:].reshape(NS, nb1, EB)
  part1 = jnp.pad(part1, ((0, 0), (0, nb0 - nb1), (0, 0)))
  return jnp.stack([part0, part1], axis=1).reshape(NW, nb0, EB)


def kernel(x, adj_indices, adj_values, W):
  n, d = x.shape
  e = adj_values.shape[0]

  need = -(-e // (NS * EB))
  nb1 = max(3, int(round(need * 0.20 / 3.0)) * 3)
  nb0 = max(nb1, -(-(need - nb1) // 3) * 3)
  e_pad = NS * (nb0 + nb1) * EB
  pad = e_pad - e

  cap0 = NS * nb0 * EB
  cr = jnp.pad(adj_indices[::-1], ((0, 0), (0, pad)))
  p0 = cr[:, :cap0].reshape(2, NS, nb0, EB)
  p1 = cr[:, cap0:].reshape(2, NS, nb1, EB)
  p1 = jnp.pad(p1, ((0, 0), (0, 0), (0, nb0 - nb1), (0, 0)))
  edata = jnp.stack([p0, p1], axis=2).transpose(1, 2, 3, 0, 4)
  edata = edata.reshape(NW, nb0, 2, EB)
  vdata = _pack(jnp.pad(adj_values, (0, pad)), nb0, nb1).reshape(
      NW, nb0, 1, EB)

  n_pad = -(-n // (NS * ZC)) * (NS * ZC)
  p = _sc_spmm(x, edata, vdata, nb1, n_pad, d)
  y = _tc_add(p[0], p[1], n)
  q = _sc_spmm(y, edata, vdata, nb1, n_pad, d)
  z = _tc_addmm(q[0], q[1], W, n)
  return z


# speedup vs baseline: 2.3265x; 2.3265x over previous
import functools

import jax
import jax.numpy as jnp
from jax import lax
from jax.experimental import pallas as pl
from jax.experimental.pallas import tpu as pltpu
from jax.experimental.pallas import tpu_sc as plsc

NC = 2
NS = 16
NW = NC * NS
LANES = 16
EB = 112
ZC = 80


def _sc_spmm(src, edata, vdata, nb1, n_pad, d):
  nb0 = edata.shape[1]
  rows_per_tile = n_pad // NS
  assert rows_per_tile % ZC == 0
  mesh = plsc.VectorSubcoreMesh(core_axis_name="c", subcore_axis_name="s")

  @functools.partial(
      pl.kernel,
      out_type=jax.ShapeDtypeStruct((NC, n_pad, d), jnp.float32),
      mesh=mesh,
      scratch_types=[
          pltpu.VMEM((3, 2, EB), jnp.int32),
          pltpu.VMEM((3, 1, EB), jnp.float32),
          pltpu.VMEM((3, EB), jnp.int32),
          pltpu.VMEM((3, EB, d), jnp.float32),
          pltpu.VMEM_SHARED((n_pad, d), jnp.float32),
          pltpu.SemaphoreType.DMA,
          pltpu.SemaphoreType.DMA,
          pltpu.SemaphoreType.DMA,
          pltpu.SemaphoreType.DMA,
          pltpu.SemaphoreType.DMA,
          pltpu.SemaphoreType.DMA,
          pltpu.SemaphoreType.DMA,
      ],
  )
  def k(src_hbm, edata_hbm, vdata_hbm, out_hbm, ebuf, vbuf, ridx, rbuf,
        acc_sh, sem_i, sg0, sg1, sg2, ss0, ss1, ss2):
    sem_g = (sg0, sg1, sg2)
    sem_s = (ss0, ss1, ss2)
    cid = lax.axis_index("c")
    sid = lax.axis_index("s")
    wid = sid * NC + cid
    cnt = jnp.where(cid == 0, nb0, nb1)

    for b in range(2):
      pltpu.sync_copy(edata_hbm.at[wid, b], ebuf.at[b])
      pltpu.sync_copy(vdata_hbm.at[wid, b], vbuf.at[b])
      pltpu.async_copy(src_hbm.at[ebuf.at[b, 0]], rbuf.at[b], sem_g[b])

    def zfill(r, _):
      for cb in range(d // LANES):
        rbuf[2, r, pl.ds(cb * LANES, LANES)] = jnp.zeros((LANES,), jnp.float32)
      return 0
    lax.fori_loop(0, ZC, zfill, 0)

    def zcopy(i, _):
      pltpu.sync_copy(
          rbuf.at[2, pl.ds(0, ZC)],
          acc_sh.at[pl.ds(sid * rows_per_tile + i * ZC, ZC)])
      return 0
    lax.fori_loop(0, rows_per_tile // ZC, zcopy, 0)
    plsc.subcore_barrier()

    def tri(t3, _):
      for b in range(3):
        t = t3 * 3 + b
        p = b
        f = (b + 2) % 3
        has_next = t + 2 < cnt

        @pl.when(has_next)
        def _():
          pltpu.async_copy(edata_hbm.at[wid, t + 2], ebuf.at[f], sem_i)
          pltpu.async_copy(vdata_hbm.at[wid, t + 2], vbuf.at[f], sem_i)

        pltpu.make_async_copy(
            src_hbm.at[ebuf.at[p, 0]], rbuf.at[p], sem_g[p]).wait()

        def scale(g, _):
          sl16 = pl.ds(g * LANES, LANES)
          ridx[p, sl16] = ebuf[p, 1, sl16]
          vv = vbuf[p, 0, sl16]
          for i in range(LANES):
            e = g * LANES + i
            v = vv[i]
            for cb in range(d // LANES):
              sl = pl.ds(cb * LANES, LANES)
              rbuf[p, e, sl] = rbuf[p, e, sl] * v
          return 0
        lax.fori_loop(0, EB // LANES, scale, 0)

        pltpu.async_copy(rbuf.at[p], acc_sh.at[ridx.at[p]], sem_s[p],
                         add=True)

        @pl.when(t >= 1)
        def _():
          pltpu.make_async_copy(
              rbuf.at[f], acc_sh.at[ridx.at[f]], sem_s[f]).wait()

        @pl.when(has_next)
        def _():
          pltpu.make_async_copy(
              edata_hbm.at[wid, t + 2], ebuf.at[f], sem_i).wait()
          pltpu.make_async_copy(
              vdata_hbm.at[wid, t + 2], vbuf.at[f], sem_i).wait()
          pltpu.async_copy(src_hbm.at[ebuf.at[f, 0]], rbuf.at[f], sem_g[f])
      return 0
    lax.fori_loop(0, cnt // 3, tri, 0)

    pltpu.make_async_copy(
        rbuf.at[2], acc_sh.at[ridx.at[2]], sem_s[2]).wait()
    plsc.subcore_barrier()

    pltpu.sync_copy(
        acc_sh.at[pl.ds(sid * rows_per_tile, rows_per_tile)],
        out_hbm.at[cid, pl.ds(sid * rows_per_tile, rows_per_tile)])

  return k(src, edata, vdata)


def _tc_add(a, b, n):
  d = a.shape[1]
  blk = n // 5

  def body(a_ref, b_ref, o_ref):
    o_ref[...] = a_ref[...] + b_ref[...]

  return pl.pallas_call(
      body,
      grid=(5,),
      in_specs=[pl.BlockSpec((blk, d), lambda i: (i, 0)),
                pl.BlockSpec((blk, d), lambda i: (i, 0))],
      out_specs=pl.BlockSpec((blk, d), lambda i: (i, 0)),
      out_shape=jax.ShapeDtypeStruct((n, d), jnp.float32),
  )(a, b)


def _tc_addmm(a, b, w, n):
  d = a.shape[1]
  blk = n // 5

  def body(a_ref, b_ref, w_ref, o_ref):
    o_ref[...] = jnp.dot(a_ref[...] + b_ref[...], w_ref[...],
                         preferred_element_type=jnp.float32)

  return pl.pallas_call(
      body,
      grid=(5,),
      in_specs=[pl.BlockSpec((blk, d), lambda i: (i, 0)),
                pl.BlockSpec((blk, d), lambda i: (i, 0)),
                pl.BlockSpec((d, d), lambda i: (0, 0))],
      out_specs=pl.BlockSpec((blk, d), lambda i: (i, 0)),
      out_shape=jax.ShapeDtypeStruct((n, d), jnp.float32),
  )(a, b, w)


def _pack(flat, nb0, nb1):
  cap0 = NS * nb0 * EB
  part0 = flat[:cap0].reshape(NS, nb0, EB)
  part1 = flat[cap0:].reshape(NS, nb1, EB)
  part1 = jnp.pad(part1, ((0, 0), (0, nb0 - nb1), (0, 0)))
  return jnp.stack([part0, part1], axis=1).reshape(NW, nb0, EB)


def kernel(x, adj_indices, adj_values, W):
  n, d = x.shape
  e = adj_values.shape[0]

  need = -(-e // (NS * EB))
  nb1 = max(3, int(round(need * 0.20 / 3.0)) * 3)
  nb0 = max(nb1, -(-(need - nb1) // 3) * 3)
  e_pad = NS * (nb0 + nb1) * EB
  pad = e_pad - e

  cols3 = _pack(jnp.pad(adj_indices[1], (0, pad)), nb0, nb1)
  rows3 = _pack(jnp.pad(adj_indices[0], (0, pad)), nb0, nb1)
  vals3 = _pack(jnp.pad(adj_values, (0, pad)), nb0, nb1)
  edata = jnp.stack([cols3, rows3], axis=2)
  vdata = vals3.reshape(NW, nb0, 1, EB)

  n_pad = -(-n // (NS * ZC)) * (NS * ZC)
  p = _sc_spmm(x, edata, vdata, nb1, n_pad, d)
  y = _tc_add(p[0], p[1], n)
  q = _sc_spmm(y, edata, vdata, nb1, n_pad, d)
  z = _tc_addmm(q[0], q[1], W, n)
  return z
